# baseline (speedup 1.0000x reference)
import jax
import jax.numpy as jnp
from jax import lax
from jax.experimental import pallas as pl
from jax.experimental.pallas import tpu as pltpu
from jax.experimental.pallas import tpu_sc as plsc

N = 10000
E = 320000
D = 128
R = 16
B = 4

NC = 2
NS = 16
H = D // NC
CH = 128
NCHUNK = 160
EPAD = NS * NCHUNK * CH
NP = 10240
RPT = NP // NS
NBUF = 2


def _mm_body(x_ref, w_ref, comp_ref, z_ref, zroot_ref):
    xb = x_ref[...]
    ys = [jnp.dot(xb, w_ref[b], preferred_element_type=jnp.float32,
                  precision=lax.Precision.HIGHEST)
          for b in range(B)]
    zroot_ref[...] = jnp.dot(xb, w_ref[B], preferred_element_type=jnp.float32,
                             precision=lax.Precision.HIGHEST)
    accs = []
    for r in range(R):
        acc = comp_ref[r, 0] * ys[0]
        for b in range(1, B):
            acc = acc + comp_ref[r, b] * ys[b]
        accs.append(acc)
    for h in range(NC):
        for r in range(0, R, 2):
            blkv = jnp.concatenate(
                [accs[r][:, h * H:(h + 1) * H],
                 accs[r + 1][:, h * H:(h + 1) * H]], axis=1)
            z_ref[h, :, r * H:(r + 2) * H] = blkv


def _relation_transform(x, weight, comp):
    blk = 1000
    grid = (N // blk,)
    return pl.pallas_call(
        _mm_body,
        grid=grid,
        in_specs=[
            pl.BlockSpec((blk, D), lambda i: (i, 0)),
            pl.BlockSpec((B + 1, D, D), lambda i: (0, 0, 0)),
            pl.BlockSpec(memory_space=pltpu.SMEM),
        ],
        out_specs=[
            pl.BlockSpec((NC, blk, R * H), lambda i: (0, i, 0)),
            pl.BlockSpec((blk, D), lambda i: (i, 0)),
        ],
        out_shape=[
            jax.ShapeDtypeStruct((NC, N, R * H), jnp.float32),
            jax.ShapeDtypeStruct((N, D), jnp.float32),
        ],
    )(x, weight, comp)


_EROWS = E // CH
_PROWS = EPAD // CH


def _pad_body(ei_s_ref, ei_d_ref, et_ref, so_ref, do_ref, to_ref):
    i = pl.program_id(0)
    row = i * CH + jax.lax.broadcasted_iota(jnp.int32, (CH, CH), 0)
    valid = row < _EROWS
    so_ref[...] = jnp.where(valid, ei_s_ref[0], 0)
    do_ref[...] = jnp.where(valid, ei_d_ref[0], NP - 1)
    to_ref[...] = jnp.where(valid, et_ref[...], 0)


def _pad_edges(edge_index, edge_type):
    ei3 = edge_index.reshape(2, _EROWS, CH)
    et2 = edge_type.reshape(_EROWS, CH)
    grid = (_PROWS // CH,)
    return pl.pallas_call(
        _pad_body,
        grid=grid,
        in_specs=[
            pl.BlockSpec((1, CH, CH), lambda i: (0, i, 0)),
            pl.BlockSpec((1, CH, CH), lambda i: (1, i, 0)),
            pl.BlockSpec((CH, CH), lambda i: (i, 0)),
        ],
        out_specs=[
            pl.BlockSpec((CH, CH), lambda i: (i, 0)),
            pl.BlockSpec((CH, CH), lambda i: (i, 0)),
            pl.BlockSpec((CH, CH), lambda i: (i, 0)),
        ],
        out_shape=[
            jax.ShapeDtypeStruct((_PROWS, CH), jnp.int32),
            jax.ShapeDtypeStruct((_PROWS, CH), jnp.int32),
            jax.ShapeDtypeStruct((_PROWS, CH), jnp.int32),
        ],
    )(ei3, ei3, et2)


def _sc_body(zmsg, src3, typ3, dst3, zrows,
             out_p, out_d,
             srcb, typb, dstb, b0, b1, degv,
             acc,
             sg0, sg1, ss0, ss1):
    bufs = (b0, b1)
    semg = (sg0, sg1)
    sems = (ss0, ss1)
    c = lax.axis_index("c")
    s = lax.axis_index("s")
    cbase = c * (N * R)

    pltpu.sync_copy(zrows, acc.at[pl.ds(s * RPT, RPT)])

    def zdeg(i, carry):
        degv[pl.ds(i * 16, 16)] = jnp.zeros((16,), jnp.float32)
        return carry
    lax.fori_loop(0, NP // 16, zdeg, 0)

    pltpu.sync_copy(src3.at[s], srcb)
    pltpu.sync_copy(typ3.at[s], typb)
    pltpu.sync_copy(dst3.at[s], dstb)
    plsc.subcore_barrier()

    def cidx(j, carry):
        for k in range(8):
            sl = pl.ds(k * 16, 16)
            srcb[j, sl] = srcb[j, sl] * R + typb[j, sl] + cbase
        return carry
    lax.fori_loop(0, NCHUNK, cidx, 0)

    for k in range(NBUF):
        pltpu.async_copy(zmsg.at[srcb.at[k]], bufs[k], semg[k])

    def step(i, carry):
        for k in range(NBUF):
            j = i * NBUF + k
            buf = bufs[k]
            pltpu.make_async_copy(zmsg.at[srcb.at[j]], buf, semg[k]).wait()
            pltpu.async_copy(buf, acc.at[dstb.at[j]], sems[k], add=True)

            @pl.when(c == 0)
            def _():
                for q in range(CH // 16):
                    dv = dstb[j, pl.ds(q * 16, 16)]
                    plsc.addupdate_scatter(degv, [dv],
                                           jnp.ones((16,), jnp.float32))

            @pl.when(j + NBUF < NCHUNK)
            def _():
                pltpu.make_async_copy(buf, acc.at[dstb.at[j]], sems[k]).wait()
                pltpu.async_copy(zmsg.at[srcb.at[j + NBUF]], buf, semg[k])
        return carry
    lax.fori_loop(0, NCHUNK // NBUF, step, 0)

    for k in range(NBUF):
        pltpu.make_async_copy(bufs[k], acc.at[dstb.at[0]], sems[k]).wait()

    plsc.subcore_barrier()
    pltpu.sync_copy(acc.at[pl.ds(s * RPT, RPT)],
                    out_p.at[c].at[pl.ds(s * RPT, RPT)])

    @pl.when(c == 0)
    def _():
        pltpu.sync_copy(degv, out_d.at[s])


def _sc_aggregate(zmsg, src3, typ3, dst3, zrows):
    mesh = plsc.VectorSubcoreMesh(core_axis_name="c", subcore_axis_name="s")
    kern = pl.kernel(
        _sc_body,
        out_type=(
            jax.ShapeDtypeStruct((NC, NP, H), jnp.float32),
            jax.ShapeDtypeStruct((NS, NP), jnp.float32),
        ),
        mesh=mesh,
        scratch_types=(
            pltpu.VMEM((NCHUNK, CH), jnp.int32),
            pltpu.VMEM((NCHUNK, CH), jnp.int32),
            pltpu.VMEM((NCHUNK, CH), jnp.int32),
            pltpu.VMEM((CH, H), jnp.float32),
            pltpu.VMEM((CH, H), jnp.float32),
            pltpu.VMEM((NP,), jnp.float32),
            pltpu.VMEM_SHARED((NP, H), jnp.float32),
            pltpu.SemaphoreType.DMA,
            pltpu.SemaphoreType.DMA,
            pltpu.SemaphoreType.DMA,
            pltpu.SemaphoreType.DMA,
        ),
        compiler_params=pltpu.CompilerParams(use_tc_tiling_on_sc=False,
                                            needs_layout_passes=False),
    )
    return kern(zmsg, src3, typ3, dst3, zrows)


def _ep_body(p0_ref, p1_ref, d_ref, zroot_ref, bias_ref, out_ref):
    deg = jnp.sum(d_ref[...], axis=0)
    inv = 1.0 / jnp.maximum(deg, 1.0)
    msg = jnp.concatenate([p0_ref[0], p1_ref[0]], axis=1)
    out_ref[...] = msg * inv + zroot_ref[...] + bias_ref[...]


def _epilogue(partials, degp, zroot, bias2d):
    blk = 1000
    grid = (N // blk,)
    return pl.pallas_call(
        _ep_body,
        grid=grid,
        in_specs=[
            pl.BlockSpec((1, blk, H), lambda i: (0, i, 0)),
            pl.BlockSpec((1, blk, H), lambda i: (1, i, 0)),
            pl.BlockSpec((NS, blk, 1), lambda i: (0, i, 0)),
            pl.BlockSpec((blk, D), lambda i: (i, 0)),
            pl.BlockSpec((1, D), lambda i: (0, 0)),
        ],
        out_specs=pl.BlockSpec((blk, D), lambda i: (i, 0)),
        out_shape=jax.ShapeDtypeStruct((N, D), jnp.float32),
    )(partials, partials, degp, zroot, bias2d)


@jax.jit
def kernel(x, edge_index, edge_type, weight, comp, bias):
    z2, zroot = _relation_transform(x, weight, comp)
    zmsg = z2.reshape(NC * N * R, H)

    srcp, dstp, typp = _pad_edges(edge_index, edge_type)
    src3 = srcp.reshape(NS, NCHUNK, CH)
    dst3 = dstp.reshape(NS, NCHUNK, CH)
    typ3 = typp.reshape(NS, NCHUNK, CH)

    zrows = jnp.zeros((RPT, H), jnp.float32)

    partials, degp = _sc_aggregate(zmsg, src3, typ3, dst3, zrows)

    return _epilogue(partials, degp.reshape(NS, NP, 1), zroot,
                     bias.reshape(1, D))


# speedup vs baseline: 1.1660x; 1.1660x over previous
import jax
import jax.numpy as jnp
from jax import lax
from jax.experimental import pallas as pl
from jax.experimental.pallas import tpu as pltpu
from jax.experimental.pallas import tpu_sc as plsc

N = 10000
E = 320000
D = 128
R = 16
B = 4

NC = 2
NS = 16
H = D // NC
CH = 128
NCHUNK = 160
EPAD = NS * NCHUNK * CH
NP = 10240
RPT = NP // NS
NBUF = 2


_MBLK = 1000


def _mm_body(x_ref, w_ref, comp_ref, z_ref, zroot_ref):
    xb = x_ref[...]
    ys = [jnp.dot(xb, w_ref[b], preferred_element_type=jnp.float32,
                  precision=lax.Precision.HIGHEST)
          for b in range(B)]
    zroot_ref[...] = jnp.dot(xb, w_ref[B], preferred_element_type=jnp.float32,
                             precision=lax.Precision.HIGHEST)
    accs = []
    for r in range(R):
        acc = comp_ref[r, 0] * ys[0]
        for b in range(1, B):
            acc = acc + comp_ref[r, b] * ys[b]
        accs.append(acc)
    for h in range(NC):
        for r in range(0, R, 2):
            blkv = jnp.concatenate(
                [accs[r][:, h * H:(h + 1) * H],
                 accs[r + 1][:, h * H:(h + 1) * H]], axis=1)
            z_ref[h, :, r * H:(r + 2) * H] = blkv


def _relation_transform(x, weight, comp):
    grid = (N // _MBLK,)
    return pl.pallas_call(
        _mm_body,
        grid=grid,
        in_specs=[
            pl.BlockSpec((_MBLK, D), lambda i: (i, 0)),
            pl.BlockSpec((B + 1, D, D), lambda i: (0, 0, 0)),
            pl.BlockSpec(memory_space=pltpu.SMEM),
        ],
        out_specs=[
            pl.BlockSpec((NC, _MBLK, R * H), lambda i: (0, i, 0)),
            pl.BlockSpec((_MBLK, D), lambda i: (i, 0)),
        ],
        out_shape=[
            jax.ShapeDtypeStruct((NC, N, R * H), jnp.float32),
            jax.ShapeDtypeStruct((N, D), jnp.float32),
        ],
    )(x, weight, comp)


_EROWS = E // CH
_PROWS = EPAD // CH


def _pad_body(ei_s_ref, ei_d_ref, et_ref, so_ref, do_ref, to_ref):
    i = pl.program_id(0)
    row = i * CH + jax.lax.broadcasted_iota(jnp.int32, (CH, CH), 0)
    valid = row < _EROWS
    so_ref[...] = jnp.where(valid, ei_s_ref[0], 0)
    do_ref[...] = jnp.where(valid, ei_d_ref[0], NP - 1)
    to_ref[...] = jnp.where(valid, et_ref[...], 0)


def _pad_edges(edge_index, edge_type):
    ei3 = edge_index.reshape(2, _EROWS, CH)
    et2 = edge_type.reshape(_EROWS, CH)
    grid = (_PROWS // CH,)
    return pl.pallas_call(
        _pad_body,
        grid=grid,
        in_specs=[
            pl.BlockSpec((1, CH, CH), lambda i: (0, i, 0)),
            pl.BlockSpec((1, CH, CH), lambda i: (1, i, 0)),
            pl.BlockSpec((CH, CH), lambda i: (i, 0)),
        ],
        out_specs=[
            pl.BlockSpec((CH, CH), lambda i: (i, 0)),
            pl.BlockSpec((CH, CH), lambda i: (i, 0)),
            pl.BlockSpec((CH, CH), lambda i: (i, 0)),
        ],
        out_shape=[
            jax.ShapeDtypeStruct((_PROWS, CH), jnp.int32),
            jax.ShapeDtypeStruct((_PROWS, CH), jnp.int32),
            jax.ShapeDtypeStruct((_PROWS, CH), jnp.int32),
        ],
    )(ei3, ei3, et2)


def _sc_body(zmsg, src3, typ3, dst3, zrows, zdeg, ones,
             out_p, out_d,
             srcb, typb, dstb, b0, b1, onesb,
             acc, dacc,
             sg0, sg1, ss0, ss1, semd):
    bufs = (b0, b1)
    semg = (sg0, sg1)
    sems = (ss0, ss1)
    c = lax.axis_index("c")
    s = lax.axis_index("s")
    cbase = c * (N * R)

    pltpu.sync_copy(zrows, acc.at[pl.ds(s * RPT, RPT)])
    pltpu.sync_copy(zdeg, dacc.at[pl.ds(s * RPT, RPT)])

    pltpu.sync_copy(ones, onesb)
    pltpu.sync_copy(src3.at[s], srcb)
    pltpu.sync_copy(typ3.at[s], typb)
    pltpu.sync_copy(dst3.at[s], dstb)
    plsc.subcore_barrier()

    def cidx(j, carry):
        for k in range(8):
            sl = pl.ds(k * 16, 16)
            srcb[j, sl] = srcb[j, sl] * R + typb[j, sl] + cbase
        return carry
    lax.fori_loop(0, NCHUNK, cidx, 0)

    for k in range(NBUF):
        pltpu.async_copy(zmsg.at[srcb.at[k]], bufs[k], semg[k])

    def step(i, carry):
        for k in range(NBUF):
            j = i * NBUF + k
            buf = bufs[k]
            pltpu.make_async_copy(zmsg.at[srcb.at[j]], buf, semg[k]).wait()
            pltpu.async_copy(buf, acc.at[dstb.at[j]], sems[k], add=True)
            pltpu.make_async_copy(buf, acc.at[dstb.at[j]], sems[k]).wait()

            @pl.when(c == k % NC)
            def _():
                pltpu.sync_copy(onesb, dacc.at[dstb.at[j]], add=True)

            @pl.when(j + NBUF < NCHUNK)
            def _():
                pltpu.async_copy(zmsg.at[srcb.at[j + NBUF]], buf, semg[k])
        return carry
    lax.fori_loop(0, NCHUNK // NBUF, step, 0)

    plsc.subcore_barrier()
    pltpu.sync_copy(acc.at[pl.ds(s * RPT, RPT)],
                    out_p.at[c].at[pl.ds(s * RPT, RPT)])

    pltpu.sync_copy(dacc.at[pl.ds(s * RPT, RPT)],
                    out_d.at[c].at[pl.ds(s * RPT, RPT)])


def _sc_aggregate(zmsg, src3, typ3, dst3, zrows, zdeg, ones):
    mesh = plsc.VectorSubcoreMesh(core_axis_name="c", subcore_axis_name="s")
    kern = pl.kernel(
        _sc_body,
        out_type=(
            jax.ShapeDtypeStruct((NC, NP, H), jnp.float32),
            jax.ShapeDtypeStruct((NC, NP, 16), jnp.float32),
        ),
        mesh=mesh,
        scratch_types=(
            pltpu.VMEM((NCHUNK, CH), jnp.int32),
            pltpu.VMEM((NCHUNK, CH), jnp.int32),
            pltpu.VMEM((NCHUNK, CH), jnp.int32),
            pltpu.VMEM((CH, H), jnp.float32),
            pltpu.VMEM((CH, H), jnp.float32),
            pltpu.VMEM((CH, 16), jnp.float32),
            pltpu.VMEM_SHARED((NP, H), jnp.float32),
            pltpu.VMEM_SHARED((NP, 16), jnp.float32),
            pltpu.SemaphoreType.DMA,
            pltpu.SemaphoreType.DMA,
            pltpu.SemaphoreType.DMA,
            pltpu.SemaphoreType.DMA,
            pltpu.SemaphoreType.DMA,
        ),
        compiler_params=pltpu.CompilerParams(use_tc_tiling_on_sc=False),
    )
    return kern(zmsg, src3, typ3, dst3, zrows, zdeg, ones)


def _ep_body(p0_ref, p1_ref, d0_ref, d1_ref, zroot_ref, bias_ref, out_ref):
    deg = d0_ref[0, :, 0:1] + d1_ref[0, :, 0:1]
    inv = 1.0 / jnp.maximum(deg, 1.0)
    msg = jnp.concatenate([p0_ref[0], p1_ref[0]], axis=1)
    out_ref[...] = msg * inv + zroot_ref[...] + bias_ref[...]


def _epilogue(partials, degp, zroot, bias2d):
    blk = 1000
    grid = (N // blk,)
    return pl.pallas_call(
        _ep_body,
        grid=grid,
        in_specs=[
            pl.BlockSpec((1, blk, H), lambda i: (0, i, 0)),
            pl.BlockSpec((1, blk, H), lambda i: (1, i, 0)),
            pl.BlockSpec((1, blk, 16), lambda i: (0, i, 0)),
            pl.BlockSpec((1, blk, 16), lambda i: (1, i, 0)),
            pl.BlockSpec((blk, D), lambda i: (i, 0)),
            pl.BlockSpec((1, D), lambda i: (0, 0)),
        ],
        out_specs=pl.BlockSpec((blk, D), lambda i: (i, 0)),
        out_shape=jax.ShapeDtypeStruct((N, D), jnp.float32),
    )(partials, partials, degp, degp, zroot, bias2d)


@jax.jit
def kernel(x, edge_index, edge_type, weight, comp, bias):
    z2, zroot = _relation_transform(x, weight, comp)
    zmsg = z2.reshape(NC * N * R, H)

    src3_, dst3_, typ3_ = _pad_edges(edge_index, edge_type)
    src3 = src3_.reshape(NS, NCHUNK, CH)
    dst3 = dst3_.reshape(NS, NCHUNK, CH)
    typ3 = typ3_.reshape(NS, NCHUNK, CH)

    zrows = jnp.zeros((RPT, H), jnp.float32)
    zdeg = jnp.zeros((RPT, 16), jnp.float32)
    ones = jnp.ones((CH, 16), jnp.float32)

    partials, degp = _sc_aggregate(zmsg, src3, typ3, dst3, zrows, zdeg, ones)

    return _epilogue(partials, degp, zroot, bias.reshape(1, D))
